# baseline (speedup 1.0000x reference)
import functools

import jax
import jax.numpy as jnp
from jax import lax
from jax.experimental import pallas as pl
from jax.experimental.pallas import tpu as pltpu
from jax.experimental.pallas import tpu_sc as plsc

_N = 10000
_E = 320000
_H = 128
_OUT = 40
_L = 3
_RES = 3
_CODES = 16

_NC = 2
_NS = 16
_NW = _NC * _NS
_EPW = _E // _NW
_CHUNK = 80
_NB = 3
_NCH = _EPW // _CHUNK
_STRIPE = _N // _NS


def _stripe_pieces():
    out, off = [], 0
    while off < _STRIPE:
        size = min(_CHUNK, _STRIPE - off)
        out.append((off, size))
        off += size
    return out

_BN = 1000
_GRID = _N // _BN

_F32 = jnp.float32
_I32 = jnp.int32


_NGRP = _NCH // _NB
_NTAIL = _NCH - _NGRP * _NB


def _sc_body(h_hbm, src_hbm, dst_hbm, z128_hbm, agg_hbm,
             src_v, dst_v, b0, b1, b2, agg_sh,
             g0, g1, g2, s0, s1, s2):
    bufs = (b0, b1, b2)
    gsem = (g0, g1, g2)
    ssem = (s0, s1, s2)
    cid = lax.axis_index("c")
    sid = lax.axis_index("s")
    wid = sid * _NC + cid
    base = sid * _STRIPE

    pltpu.sync_copy(src_hbm.at[wid], src_v)
    pltpu.sync_copy(dst_hbm.at[wid], dst_v)

    pltpu.sync_copy(z128_hbm, b0)
    for off, size in _stripe_pieces():
        pltpu.sync_copy(b0.at[pl.ds(0, size)],
                        agg_sh.at[pl.ds(base + off, size)])
    plsc.subcore_barrier()

    for b in range(_NB):
        pltpu.async_copy(h_hbm.at[src_v.at[b]], bufs[b], gsem[b])

    def group(p, carry):
        a = p * _NB
        for b in range(_NB):
            pltpu.make_async_copy(h_hbm.at[src_v.at[a + b]],
                                  bufs[b], gsem[b]).wait()
            pltpu.async_copy(bufs[b], agg_sh.at[dst_v.at[a + b]],
                             ssem[b], add=True)
        for b in range(_NB):
            nxt = jnp.minimum(a + _NB + b, _NCH - 1)
            pltpu.make_async_copy(bufs[b], agg_sh.at[dst_v.at[a + b]],
                                  ssem[b]).wait()
            pltpu.async_copy(h_hbm.at[src_v.at[nxt]], bufs[b], gsem[b])
        return carry

    lax.fori_loop(0, _NGRP - 1, group, 0)
    last = (_NGRP - 1) * _NB
    for b in range(_NB):
        j = last + b
        pltpu.make_async_copy(h_hbm.at[src_v.at[j]], bufs[b], gsem[b]).wait()
        pltpu.sync_copy(bufs[b], agg_sh.at[dst_v.at[j]], add=True)
        nxt = min(last + _NB + b, _NCH - 1)
        pltpu.async_copy(h_hbm.at[src_v.at[nxt]], bufs[b], gsem[b])
    for b in range(_NB):
        j = last + _NB + b
        pltpu.make_async_copy(h_hbm.at[src_v.at[_NCH - 1]],
                              bufs[b], gsem[b]).wait()
        if j <= _NCH - 1:
            pltpu.sync_copy(bufs[b], agg_sh.at[dst_v.at[j]], add=True)
    plsc.subcore_barrier()

    for off, size in _stripe_pieces():
        sl = pl.ds(base + off, size)
        pltpu.sync_copy(agg_sh.at[sl], b0.at[pl.ds(0, size)])
        pltpu.sync_copy(b0.at[pl.ds(0, size)], agg_hbm.at[cid].at[sl])


def _make_sc_call():
    mesh = plsc.VectorSubcoreMesh(
        core_axis_name="c", subcore_axis_name="s",
        num_cores=_NC, num_subcores=_NS)
    scratch = [
        pltpu.VMEM((_NCH, _CHUNK), _I32),
        pltpu.VMEM((_NCH, _CHUNK), _I32),
        pltpu.VMEM((_CHUNK, _H), _F32),
        pltpu.VMEM((_CHUNK, _H), _F32),
        pltpu.VMEM((_CHUNK, _H), _F32),
        pltpu.VMEM_SHARED((_N, _H), _F32),
        pltpu.SemaphoreType.DMA,
        pltpu.SemaphoreType.DMA,
        pltpu.SemaphoreType.DMA,
        pltpu.SemaphoreType.DMA,
        pltpu.SemaphoreType.DMA,
        pltpu.SemaphoreType.DMA,
    ]
    return pl.kernel(
        _sc_body,
        out_type=jax.ShapeDtypeStruct((_NC, _N, _H), _F32),
        mesh=mesh,
        scratch_types=scratch,
        compiler_params=pltpu.CompilerParams(use_tc_tiling_on_sc=False),
    )


def _deg_body(dst_hbm, one16_hbm, z16_hbm, deg_hbm,
              dst_v, ones_v, degrows_v, deg_sh, sem):
    cid = lax.axis_index("c")
    sid = lax.axis_index("s")
    wid = sid * _NC + cid
    base = sid * _STRIPE

    pltpu.sync_copy(dst_hbm.at[wid], dst_v)
    pltpu.sync_copy(one16_hbm, ones_v)
    pltpu.sync_copy(z16_hbm, degrows_v)
    for off, size in _stripe_pieces():
        pltpu.sync_copy(degrows_v.at[pl.ds(0, size)],
                        deg_sh.at[pl.ds(base + off, size)])
    plsc.subcore_barrier()

    def fire(j, carry):
        pltpu.async_copy(ones_v, deg_sh.at[dst_v.at[j]], sem, add=True)
        return carry

    lax.fori_loop(0, _NCH, fire, 0)

    def drain(j, carry):
        pltpu.make_async_copy(ones_v, deg_sh.at[dst_v.at[0]], sem).wait()
        return carry

    lax.fori_loop(0, _NCH, drain, 0)
    plsc.subcore_barrier()

    for off, size in _stripe_pieces():
        sl = pl.ds(base + off, size)
        pltpu.sync_copy(deg_sh.at[sl], degrows_v.at[pl.ds(0, size)])
        pltpu.sync_copy(degrows_v.at[pl.ds(0, size)], deg_hbm.at[cid].at[sl])


def _make_deg_call():
    mesh = plsc.VectorSubcoreMesh(
        core_axis_name="c", subcore_axis_name="s",
        num_cores=_NC, num_subcores=_NS)
    scratch = [
        pltpu.VMEM((_NCH, _CHUNK), _I32),
        pltpu.VMEM((_CHUNK, 16), _F32),
        pltpu.VMEM((_CHUNK, 16), _F32),
        pltpu.VMEM_SHARED((_N, 16), _F32),
        pltpu.SemaphoreType.DMA,
    ]
    return pl.kernel(
        _deg_body,
        out_type=jax.ShapeDtypeStruct((_NC, _N, 16), _F32),
        mesh=mesh,
        scratch_types=scratch,
        compiler_params=pltpu.CompilerParams(use_tc_tiling_on_sc=False),
    )


def _mmT(a, w):
    return lax.dot_general(
        a, w, (((1,), (1,)), ((), ())),
        preferred_element_type=_F32, precision=lax.Precision.DEFAULT)


def _rownorm(v):
    ss = jnp.sum(v * v, axis=1, keepdims=True)
    return v / jnp.maximum(jnp.sqrt(ss), 1e-12)


def _tc_layer_body(final, *refs):
    if final:
        (h_ref, xl_ref, agg_ref, deg_ref, wl_ref, bl_ref, wr_ref, wlin_ref,
         blin_ref, g_ref, b_ref, cb_ref, wp_ref, bp_ref, wg_ref, bg_ref,
         h_out, xl_out, ids_out, loss_ref, pred_out, gnn_out) = refs
    else:
        (h_ref, xl_ref, agg_ref, deg_ref, wl_ref, bl_ref, wr_ref, wlin_ref,
         blin_ref, g_ref, b_ref, cb_ref,
         h_out, xl_out, ids_out, loss_ref) = refs

    h = h_ref[...]
    agg = agg_ref[0] + agg_ref[1]
    deg = deg_ref[0, :, 0:1] + deg_ref[1, :, 0:1]
    aggn = agg * (1.0 / jnp.maximum(deg, 1.0))

    out = _mmT(aggn, wl_ref[...]) + bl_ref[...] + _mmT(h, wr_ref[...])
    z = _rownorm(out) + _mmT(h, wlin_ref[...]) + blin_ref[...]
    scale = g_ref[...] * (1.0 / jnp.sqrt(jnp.float32(1.0 + 1e-5)))
    hnew = jnp.maximum(z * scale + b_ref[...], 0.0)
    h_out[...] = hnew
    xl = xl_ref[...] + hnew
    xl_out[...] = xl

    resid = hnew
    lsum = jnp.float32(0.0)
    idcols = []
    for r in range(_RES):
        cbn = _rownorm(cb_ref[r])
        rn = _rownorm(resid)
        sim = _mmT(rn, cbn)
        m = jnp.max(sim, axis=1, keepdims=True)
        io = lax.broadcasted_iota(_I32, sim.shape, 1)
        idx = jnp.min(jnp.where(sim >= m, io, _CODES), axis=1, keepdims=True)
        oh = (io == idx).astype(_F32)
        q = lax.dot_general(
            oh, cbn, (((1,), (0,)), ((), ())),
            preferred_element_type=_F32, precision=lax.Precision.HIGHEST)
        d = q - resid
        lsum = lsum + jnp.sum(d * d)
        idcols.append(idx)
        resid = resid - q
    ids_out[...] = jnp.concatenate(idcols, axis=1)

    @pl.when(pl.program_id(0) == 0)
    def _():
        loss_ref[...] = jnp.zeros((1, 1), _F32)
    loss_ref[...] += jnp.full((1, 1), lsum * jnp.float32(0.25 / (_N * _H)))

    if final:
        pred_out[...] = _mmT(xl, wp_ref[...]) + bp_ref[...]
        gnn_out[...] = _mmT(xl, wg_ref[...]) + bg_ref[...]


def _row_spec(cols):
    return pl.BlockSpec((_BN, cols), lambda i: (i, 0))


def _const_spec(shape):
    nd = len(shape)
    return pl.BlockSpec(shape, lambda i, _n=nd: (0,) * _n)


def _make_tc_layer(final):
    in_specs = [
        _row_spec(_H),
        _row_spec(_H),
        pl.BlockSpec((_NC, _BN, _H), lambda i: (0, i, 0)),
        pl.BlockSpec((_NC, _BN, 16), lambda i: (0, i, 0)),
        _const_spec((_H, _H)),
        _const_spec((1, _H)),
        _const_spec((_H, _H)),
        _const_spec((_H, _H)),
        _const_spec((1, _H)),
        _const_spec((1, _H)),
        _const_spec((1, _H)),
        _const_spec((_RES, _CODES, _H)),
    ]
    out_shape = [
        jax.ShapeDtypeStruct((_N, _H), _F32),
        jax.ShapeDtypeStruct((_N, _H), _F32),
        jax.ShapeDtypeStruct((_N, _RES), _I32),
        jax.ShapeDtypeStruct((1, 1), _F32),
    ]
    out_specs = [
        _row_spec(_H),
        _row_spec(_H),
        _row_spec(_RES),
        pl.BlockSpec((1, 1), lambda i: (0, 0)),
    ]
    if final:
        in_specs += [
            _const_spec((_OUT, _H)),
            _const_spec((1, _OUT)),
            _const_spec((16, _H)),
            _const_spec((1, 16)),
        ]
        out_shape += [
            jax.ShapeDtypeStruct((_N, _OUT), _F32),
            jax.ShapeDtypeStruct((_N, 16), _F32),
        ]
        out_specs += [
            _row_spec(_OUT),
            _row_spec(16),
        ]
    return pl.pallas_call(
        functools.partial(_tc_layer_body, final),
        grid=(_GRID,),
        in_specs=in_specs,
        out_specs=out_specs,
        out_shape=out_shape,
    )


def kernel(x, edge_index, Wl, bl, Wr, Wlin, blin, bn_g, bn_b, codebooks,
           W_gnn, b_gnn, W_pred, b_pred):
    src = edge_index[0].reshape(_NW, _NCH, _CHUNK)
    dst = edge_index[1].reshape(_NW, _NCH, _CHUNK)
    z128 = jnp.zeros((_CHUNK, _H), _F32)
    one16 = jnp.ones((_CHUNK, 16), _F32)
    z16 = jnp.zeros((_CHUNK, 16), _F32)

    sc_agg = _make_sc_call()
    sc_deg = _make_deg_call()
    tc_mid = _make_tc_layer(False)
    tc_last = _make_tc_layer(True)

    wg_pad = jnp.zeros((16, _H), _F32).at[: _L * _RES].set(W_gnn)
    bg_pad = jnp.zeros((1, 16), _F32).at[0, : _L * _RES].set(b_gnn)

    h = x
    xl = jnp.zeros((_N, _H), _F32)
    losses = []
    ids = []
    deg2 = sc_deg(dst, one16, z16)
    for i in range(_L):
        agg2 = sc_agg(h, src, dst, z128)
        args = (h, xl, agg2, deg2, Wl[i], bl[i].reshape(1, _H), Wr[i],
                Wlin[i], blin[i].reshape(1, _H), bn_g[i].reshape(1, _H),
                bn_b[i].reshape(1, _H), codebooks[i])
        if i < _L - 1:
            h, xl, ids_i, loss_i = tc_mid(*args)
        else:
            h, xl, ids_i, loss_i, pred, gnn_pad = tc_last(
                *args, W_pred, b_pred.reshape(1, _OUT), wg_pad, bg_pad)
        losses.append(loss_i)
        ids.append(ids_i)

    total_commit = (losses[0] + losses[1] + losses[2])[0, 0]
    id_cat = jnp.concatenate(ids, axis=1)
    gnn_id = gnn_pad[:, : _L * _RES]
    return (pred, total_commit, id_cat, gnn_id)


# speedup vs baseline: 1.2247x; 1.2247x over previous
import functools

import jax
import jax.numpy as jnp
from jax import lax
from jax.experimental import pallas as pl
from jax.experimental.pallas import tpu as pltpu
from jax.experimental.pallas import tpu_sc as plsc

_N = 10000
_E = 320000
_H = 128
_OUT = 40
_L = 3
_RES = 3
_CODES = 16

_NC = 2
_NS = 16
_NW = _NC * _NS
_EPW = _E // _NW
_CHUNK = 100
_NCH = _EPW // _CHUNK
_STRIPE = _N // _NS


def _stripe_pieces():
    out, off = [], 0
    while off < _STRIPE:
        size = min(_CHUNK, _STRIPE - off)
        out.append((off, size))
        off += size
    return out

_BN = 1000
_GRID = _N // _BN

_F32 = jnp.float32
_I32 = jnp.int32


def _sc_body(h_hbm, src_hbm, dst_hbm, z128_hbm, agg_hbm,
             src_v, dst_v, b0, b1, agg_sh, g0, g1, aux):
    cid = lax.axis_index("c")
    sid = lax.axis_index("s")
    wid = sid * _NC + cid
    base = sid * _STRIPE
    pieces = _stripe_pieces()

    pltpu.async_copy(src_hbm.at[wid], src_v, aux)
    pltpu.async_copy(dst_hbm.at[wid], dst_v, aux)
    pltpu.sync_copy(z128_hbm, b0)
    for off, size in pieces:
        pltpu.async_copy(b0.at[pl.ds(0, size)],
                         agg_sh.at[pl.ds(base + off, size)], aux)
    pltpu.make_async_copy(src_hbm.at[wid], src_v, aux).wait()
    pltpu.make_async_copy(dst_hbm.at[wid], dst_v, aux).wait()
    for off, size in pieces:
        pltpu.make_async_copy(b0.at[pl.ds(0, size)],
                              agg_sh.at[pl.ds(base + off, size)], aux).wait()
    plsc.subcore_barrier()

    pltpu.async_copy(h_hbm.at[src_v.at[0]], b0, g0)

    def pair(p, carry):
        a = 2 * p
        pltpu.async_copy(h_hbm.at[src_v.at[a + 1]], b1, g1)
        pltpu.make_async_copy(h_hbm.at[src_v.at[a]], b0, g0).wait()
        pltpu.sync_copy(b0, agg_sh.at[dst_v.at[a]], add=True)
        nxt = jnp.minimum(a + 2, _NCH - 1)
        pltpu.async_copy(h_hbm.at[src_v.at[nxt]], b0, g0)
        pltpu.make_async_copy(h_hbm.at[src_v.at[a + 1]], b1, g1).wait()
        pltpu.sync_copy(b1, agg_sh.at[dst_v.at[a + 1]], add=True)
        return carry

    lax.fori_loop(0, _NCH // 2, pair, 0)
    pltpu.make_async_copy(h_hbm.at[src_v.at[_NCH - 1]], b0, g0).wait()
    if _NCH % 2 == 1:
        pltpu.sync_copy(b0, agg_sh.at[dst_v.at[_NCH - 1]], add=True)
    plsc.subcore_barrier()

    pend = [None, None]
    for i, (off, size) in enumerate(pieces):
        par = i % 2
        buf, s = (b0, g0) if par == 0 else (b1, g1)
        if pend[par] is not None:
            poff, psize = pend[par]
            pltpu.make_async_copy(
                buf.at[pl.ds(0, psize)],
                agg_hbm.at[cid].at[pl.ds(base + poff, psize)], s).wait()
        pltpu.sync_copy(agg_sh.at[pl.ds(base + off, size)],
                        buf.at[pl.ds(0, size)])
        pltpu.async_copy(buf.at[pl.ds(0, size)],
                         agg_hbm.at[cid].at[pl.ds(base + off, size)], s)
        pend[par] = (off, size)
    for par in (0, 1):
        if pend[par] is not None:
            poff, psize = pend[par]
            buf, s = (b0, g0) if par == 0 else (b1, g1)
            pltpu.make_async_copy(
                buf.at[pl.ds(0, psize)],
                agg_hbm.at[cid].at[pl.ds(base + poff, psize)], s).wait()


def _make_sc_call():
    mesh = plsc.VectorSubcoreMesh(
        core_axis_name="c", subcore_axis_name="s",
        num_cores=_NC, num_subcores=_NS)
    scratch = [
        pltpu.VMEM((_NCH, _CHUNK), _I32),
        pltpu.VMEM((_NCH, _CHUNK), _I32),
        pltpu.VMEM((_CHUNK, _H), _F32),
        pltpu.VMEM((_CHUNK, _H), _F32),
        pltpu.VMEM_SHARED((_N, _H), _F32),
        pltpu.SemaphoreType.DMA,
        pltpu.SemaphoreType.DMA,
        pltpu.SemaphoreType.DMA,
    ]
    return pl.kernel(
        _sc_body,
        out_type=jax.ShapeDtypeStruct((_NC, _N, _H), _F32),
        mesh=mesh,
        scratch_types=scratch,
        compiler_params=pltpu.CompilerParams(use_tc_tiling_on_sc=False),
    )


def _deg_body(dst_hbm, one16_hbm, z16_hbm, deg_hbm,
              dst_v, ones_v, degrows_v, deg_sh, sem):
    cid = lax.axis_index("c")
    sid = lax.axis_index("s")
    wid = sid * _NC + cid
    base = sid * _STRIPE

    pltpu.sync_copy(dst_hbm.at[wid], dst_v)
    pltpu.sync_copy(one16_hbm, ones_v)
    pltpu.sync_copy(z16_hbm, degrows_v)
    for off, size in _stripe_pieces():
        pltpu.sync_copy(degrows_v.at[pl.ds(0, size)],
                        deg_sh.at[pl.ds(base + off, size)])
    plsc.subcore_barrier()

    def fire(j, carry):
        pltpu.async_copy(ones_v, deg_sh.at[dst_v.at[j]], sem, add=True)
        return carry

    lax.fori_loop(0, _NCH, fire, 0)

    def drain(j, carry):
        pltpu.make_async_copy(ones_v, deg_sh.at[dst_v.at[0]], sem).wait()
        return carry

    lax.fori_loop(0, _NCH, drain, 0)
    plsc.subcore_barrier()

    for off, size in _stripe_pieces():
        sl = pl.ds(base + off, size)
        pltpu.sync_copy(deg_sh.at[sl], degrows_v.at[pl.ds(0, size)])
        pltpu.sync_copy(degrows_v.at[pl.ds(0, size)], deg_hbm.at[cid].at[sl])


def _make_deg_call():
    mesh = plsc.VectorSubcoreMesh(
        core_axis_name="c", subcore_axis_name="s",
        num_cores=_NC, num_subcores=_NS)
    scratch = [
        pltpu.VMEM((_NCH, _CHUNK), _I32),
        pltpu.VMEM((_CHUNK, 16), _F32),
        pltpu.VMEM((_CHUNK, 16), _F32),
        pltpu.VMEM_SHARED((_N, 16), _F32),
        pltpu.SemaphoreType.DMA,
    ]
    return pl.kernel(
        _deg_body,
        out_type=jax.ShapeDtypeStruct((_NC, _N, 16), _F32),
        mesh=mesh,
        scratch_types=scratch,
        compiler_params=pltpu.CompilerParams(use_tc_tiling_on_sc=False),
    )


def _mmT(a, w):
    return lax.dot_general(
        a, w, (((1,), (1,)), ((), ())),
        preferred_element_type=_F32, precision=lax.Precision.DEFAULT)


def _rownorm(v):
    ss = jnp.sum(v * v, axis=1, keepdims=True)
    return v / jnp.maximum(jnp.sqrt(ss), 1e-12)


def _tc_update_body(h_ref, agg_ref, deg_ref, wl_ref, bl_ref, wr_ref,
                    wlin_ref, blin_ref, g_ref, b_ref, h_out):
    h = h_ref[...]
    agg = agg_ref[0] + agg_ref[1]
    deg = deg_ref[0, :, 0:1] + deg_ref[1, :, 0:1]
    aggn = agg * (1.0 / jnp.maximum(deg, 1.0))

    out = _mmT(aggn, wl_ref[...]) + bl_ref[...] + _mmT(h, wr_ref[...])
    z = _rownorm(out) + _mmT(h, wlin_ref[...]) + blin_ref[...]
    scale = g_ref[...] * (1.0 / jnp.sqrt(jnp.float32(1.0 + 1e-5)))
    h_out[...] = jnp.maximum(z * scale + b_ref[...], 0.0)


def _tc_rvq_body(first, *refs):
    if first:
        (h_ref, cb_ref, xl_out, ids_out, loss_ref) = refs
    else:
        (h_ref, xl_ref, cb_ref, xl_out, ids_out, loss_ref) = refs

    hnew = h_ref[...]
    if first:
        xl_out[...] = hnew
    else:
        xl_out[...] = xl_ref[...] + hnew

    resid = hnew
    lsum = jnp.float32(0.0)
    idcols = []
    for r in range(_RES):
        cbn = _rownorm(cb_ref[r])
        rn = _rownorm(resid)
        sim = _mmT(rn, cbn)
        m = jnp.max(sim, axis=1, keepdims=True)
        io = lax.broadcasted_iota(_I32, sim.shape, 1)
        idx = jnp.min(jnp.where(sim >= m, io, _CODES), axis=1, keepdims=True)
        oh = (io == idx).astype(_F32)
        q = lax.dot_general(
            oh, cbn, (((1,), (0,)), ((), ())),
            preferred_element_type=_F32, precision=lax.Precision.HIGHEST)
        d = q - resid
        lsum = lsum + jnp.sum(d * d)
        idcols.append(idx)
        resid = resid - q
    ids_out[...] = jnp.concatenate(idcols, axis=1)

    @pl.when(pl.program_id(0) == 0)
    def _():
        loss_ref[...] = jnp.zeros((1, 1), _F32)
    loss_ref[...] += jnp.full((1, 1), lsum * jnp.float32(0.25 / (_N * _H)))


def _tc_head_body(xl_ref, wp_ref, bp_ref, wg_ref, bg_ref, pred_out, gnn_out):
    xl = xl_ref[...]
    pred_out[...] = _mmT(xl, wp_ref[...]) + bp_ref[...]
    gnn_out[...] = _mmT(xl, wg_ref[...]) + bg_ref[...]


def _row_spec(cols):
    return pl.BlockSpec((_BN, cols), lambda i: (i, 0))


def _const_spec(shape):
    nd = len(shape)
    return pl.BlockSpec(shape, lambda i, _n=nd: (0,) * _n)


def _make_tc_update():
    return pl.pallas_call(
        _tc_update_body,
        grid=(_GRID,),
        in_specs=[
            _row_spec(_H),
            pl.BlockSpec((_NC, _BN, _H), lambda i: (0, i, 0)),
            pl.BlockSpec((_NC, _BN, 16), lambda i: (0, i, 0)),
            _const_spec((_H, _H)),
            _const_spec((1, _H)),
            _const_spec((_H, _H)),
            _const_spec((_H, _H)),
            _const_spec((1, _H)),
            _const_spec((1, _H)),
            _const_spec((1, _H)),
        ],
        out_specs=_row_spec(_H),
        out_shape=jax.ShapeDtypeStruct((_N, _H), _F32),
    )


def _make_tc_rvq(first):
    in_specs = [_row_spec(_H)]
    if not first:
        in_specs.append(_row_spec(_H))
    in_specs.append(_const_spec((_RES, _CODES, _H)))
    return pl.pallas_call(
        functools.partial(_tc_rvq_body, first),
        grid=(_GRID,),
        in_specs=in_specs,
        out_specs=[
            _row_spec(_H),
            _row_spec(_RES),
            pl.BlockSpec((1, 1), lambda i: (0, 0)),
        ],
        out_shape=[
            jax.ShapeDtypeStruct((_N, _H), _F32),
            jax.ShapeDtypeStruct((_N, _RES), _I32),
            jax.ShapeDtypeStruct((1, 1), _F32),
        ],
    )


def _make_tc_head():
    return pl.pallas_call(
        _tc_head_body,
        grid=(_GRID,),
        in_specs=[
            _row_spec(_H),
            _const_spec((_OUT, _H)),
            _const_spec((1, _OUT)),
            _const_spec((16, _H)),
            _const_spec((1, 16)),
        ],
        out_specs=[_row_spec(_OUT), _row_spec(16)],
        out_shape=[
            jax.ShapeDtypeStruct((_N, _OUT), _F32),
            jax.ShapeDtypeStruct((_N, 16), _F32),
        ],
    )


def kernel(x, edge_index, Wl, bl, Wr, Wlin, blin, bn_g, bn_b, codebooks,
           W_gnn, b_gnn, W_pred, b_pred):
    src = edge_index[0].reshape(_NW, _NCH, _CHUNK)
    dst = edge_index[1].reshape(_NW, _NCH, _CHUNK)
    z128 = jnp.zeros((_CHUNK, _H), _F32)
    one16 = jnp.ones((_CHUNK, 16), _F32)
    z16 = jnp.zeros((_CHUNK, 16), _F32)

    sc_agg = _make_sc_call()
    sc_deg = _make_deg_call()
    tc_update = _make_tc_update()
    tc_rvq_first = _make_tc_rvq(True)
    tc_rvq = _make_tc_rvq(False)
    tc_head = _make_tc_head()

    wg_pad = jnp.zeros((16, _H), _F32).at[: _L * _RES].set(W_gnn)
    bg_pad = jnp.zeros((1, 16), _F32).at[0, : _L * _RES].set(b_gnn)

    h = x
    xl = None
    losses = []
    ids = []
    deg2 = sc_deg(dst, one16, z16)
    for i in range(_L):
        agg2 = sc_agg(h, src, dst, z128)
        h = tc_update(h, agg2, deg2, Wl[i], bl[i].reshape(1, _H), Wr[i],
                      Wlin[i], blin[i].reshape(1, _H), bn_g[i].reshape(1, _H),
                      bn_b[i].reshape(1, _H))
        if i == 0:
            xl, ids_i, loss_i = tc_rvq_first(h, codebooks[i])
        else:
            xl, ids_i, loss_i = tc_rvq(h, xl, codebooks[i])
        losses.append(loss_i)
        ids.append(ids_i)
    pred, gnn_pad = tc_head(xl, W_pred, b_pred.reshape(1, _OUT), wg_pad, bg_pad)

    total_commit = (losses[0] + losses[1] + losses[2])[0, 0]
    id_cat = jnp.concatenate(ids, axis=1)
    gnn_id = gnn_pad[:, : _L * _RES]
    return (pred, total_commit, id_cat, gnn_id)


# speedup vs baseline: 1.2263x; 1.0013x over previous
import functools

import jax
import jax.numpy as jnp
from jax import lax
from jax.experimental import pallas as pl
from jax.experimental.pallas import tpu as pltpu
from jax.experimental.pallas import tpu_sc as plsc

_N = 10000
_E = 320000
_H = 128
_OUT = 40
_L = 3
_RES = 3
_CODES = 16

_NC = 2
_NS = 16
_NW = _NC * _NS
_EPW = _E // _NW
_CHUNK = 100
_NCH = _EPW // _CHUNK
_STRIPE = _N // _NS


def _stripe_pieces():
    out, off = [], 0
    while off < _STRIPE:
        size = min(_CHUNK, _STRIPE - off)
        out.append((off, size))
        off += size
    return out

_BN = 1000
_GRID = _N // _BN

_F32 = jnp.float32
_I32 = jnp.int32


def _sc_body(h_hbm, src_hbm, dst_hbm, z128_hbm, agg_hbm,
             src_v, dst_v, b0, b1, agg_sh, g0, g1, aux):
    cid = lax.axis_index("c")
    sid = lax.axis_index("s")
    wid = sid * _NC + cid
    base = sid * _STRIPE
    pieces = _stripe_pieces()

    pltpu.async_copy(src_hbm.at[wid], src_v, aux)
    pltpu.async_copy(dst_hbm.at[wid], dst_v, aux)
    pltpu.sync_copy(z128_hbm, b0)
    for off, size in pieces:
        pltpu.async_copy(b0.at[pl.ds(0, size)],
                         agg_sh.at[pl.ds(base + off, size)], aux)
    pltpu.make_async_copy(src_hbm.at[wid], src_v, aux).wait()
    pltpu.make_async_copy(dst_hbm.at[wid], dst_v, aux).wait()
    for off, size in pieces:
        pltpu.make_async_copy(b0.at[pl.ds(0, size)],
                              agg_sh.at[pl.ds(base + off, size)], aux).wait()
    plsc.subcore_barrier()

    pltpu.async_copy(h_hbm.at[src_v.at[0]], b0, g0)

    def pair(p, carry):
        a = 2 * p
        pltpu.async_copy(h_hbm.at[src_v.at[a + 1]], b1, g1)
        pltpu.make_async_copy(h_hbm.at[src_v.at[a]], b0, g0).wait()
        pltpu.sync_copy(b0, agg_sh.at[dst_v.at[a]], add=True)
        nxt = jnp.minimum(a + 2, _NCH - 1)
        pltpu.async_copy(h_hbm.at[src_v.at[nxt]], b0, g0)
        pltpu.make_async_copy(h_hbm.at[src_v.at[a + 1]], b1, g1).wait()
        pltpu.sync_copy(b1, agg_sh.at[dst_v.at[a + 1]], add=True)
        return carry

    lax.fori_loop(0, _NCH // 2, pair, 0)
    pltpu.make_async_copy(h_hbm.at[src_v.at[_NCH - 1]], b0, g0).wait()
    if _NCH % 2 == 1:
        pltpu.sync_copy(b0, agg_sh.at[dst_v.at[_NCH - 1]], add=True)
    plsc.subcore_barrier()

    pend = [None, None]
    for i, (off, size) in enumerate(pieces):
        par = i % 2
        buf, s = (b0, g0) if par == 0 else (b1, g1)
        if pend[par] is not None:
            poff, psize = pend[par]
            pltpu.make_async_copy(
                buf.at[pl.ds(0, psize)],
                agg_hbm.at[cid].at[pl.ds(base + poff, psize)], s).wait()
        pltpu.sync_copy(agg_sh.at[pl.ds(base + off, size)],
                        buf.at[pl.ds(0, size)])
        pltpu.async_copy(buf.at[pl.ds(0, size)],
                         agg_hbm.at[cid].at[pl.ds(base + off, size)], s)
        pend[par] = (off, size)
    for par in (0, 1):
        if pend[par] is not None:
            poff, psize = pend[par]
            buf, s = (b0, g0) if par == 0 else (b1, g1)
            pltpu.make_async_copy(
                buf.at[pl.ds(0, psize)],
                agg_hbm.at[cid].at[pl.ds(base + poff, psize)], s).wait()


def _make_sc_call():
    mesh = plsc.VectorSubcoreMesh(
        core_axis_name="c", subcore_axis_name="s",
        num_cores=_NC, num_subcores=_NS)
    scratch = [
        pltpu.VMEM((_NCH, _CHUNK), _I32),
        pltpu.VMEM((_NCH, _CHUNK), _I32),
        pltpu.VMEM((_CHUNK, _H), _F32),
        pltpu.VMEM((_CHUNK, _H), _F32),
        pltpu.VMEM_SHARED((_N, _H), _F32),
        pltpu.SemaphoreType.DMA,
        pltpu.SemaphoreType.DMA,
        pltpu.SemaphoreType.DMA,
    ]
    return pl.kernel(
        _sc_body,
        out_type=jax.ShapeDtypeStruct((_NC, _N, _H), _F32),
        mesh=mesh,
        scratch_types=scratch,
        compiler_params=pltpu.CompilerParams(use_tc_tiling_on_sc=False),
    )


def _deg_body(dst_hbm, one16_hbm, z16_hbm, deg_hbm,
              dst_v, ones_v, degrows_v, deg_sh, sem):
    cid = lax.axis_index("c")
    sid = lax.axis_index("s")
    wid = sid * _NC + cid
    base = sid * _STRIPE

    pltpu.sync_copy(dst_hbm.at[wid], dst_v)
    pltpu.sync_copy(one16_hbm, ones_v)
    pltpu.sync_copy(z16_hbm, degrows_v)
    for off, size in _stripe_pieces():
        pltpu.sync_copy(degrows_v.at[pl.ds(0, size)],
                        deg_sh.at[pl.ds(base + off, size)])
    plsc.subcore_barrier()

    def fire(j, carry):
        pltpu.async_copy(ones_v, deg_sh.at[dst_v.at[j]], sem, add=True)
        return carry

    lax.fori_loop(0, _NCH, fire, 0)

    def drain(j, carry):
        pltpu.make_async_copy(ones_v, deg_sh.at[dst_v.at[0]], sem).wait()
        return carry

    lax.fori_loop(0, _NCH, drain, 0)
    plsc.subcore_barrier()

    for off, size in _stripe_pieces():
        sl = pl.ds(base + off, size)
        pltpu.sync_copy(deg_sh.at[sl], degrows_v.at[pl.ds(0, size)])
        pltpu.sync_copy(degrows_v.at[pl.ds(0, size)], deg_hbm.at[cid].at[sl])


def _make_deg_call():
    mesh = plsc.VectorSubcoreMesh(
        core_axis_name="c", subcore_axis_name="s",
        num_cores=_NC, num_subcores=_NS)
    scratch = [
        pltpu.VMEM((_NCH, _CHUNK), _I32),
        pltpu.VMEM((_CHUNK, 16), _F32),
        pltpu.VMEM((_CHUNK, 16), _F32),
        pltpu.VMEM_SHARED((_N, 16), _F32),
        pltpu.SemaphoreType.DMA,
    ]
    return pl.kernel(
        _deg_body,
        out_type=jax.ShapeDtypeStruct((_NC, _N, 16), _F32),
        mesh=mesh,
        scratch_types=scratch,
        compiler_params=pltpu.CompilerParams(use_tc_tiling_on_sc=False),
    )


def _mmT(a, w):
    return lax.dot_general(
        a, w, (((1,), (1,)), ((), ())),
        preferred_element_type=_F32, precision=lax.Precision.DEFAULT)


def _rownorm(v):
    ss = jnp.sum(v * v, axis=1, keepdims=True)
    return v / jnp.maximum(jnp.sqrt(ss), 1e-12)


def _tc_update_body(h_ref, agg_ref, deg_ref, wl_ref, bl_ref, wr_ref,
                    wlin_ref, blin_ref, g_ref, b_ref, h_out):
    h = h_ref[...]
    agg = agg_ref[0] + agg_ref[1]
    deg = deg_ref[0, :, 0:1] + deg_ref[1, :, 0:1]
    aggn = agg * (1.0 / jnp.maximum(deg, 1.0))

    out = _mmT(aggn, wl_ref[0]) + bl_ref[0] + _mmT(h, wr_ref[0])
    z = _rownorm(out) + _mmT(h, wlin_ref[0]) + blin_ref[0]
    scale = g_ref[0] * (1.0 / jnp.sqrt(jnp.float32(1.0 + 1e-5)))
    h_out[...] = jnp.maximum(z * scale + b_ref[0], 0.0)


def _tc_rvq_body(first, *refs):
    if first:
        (h_ref, cb_ref, xl_out, ids_out, loss_ref) = refs
    else:
        (h_ref, xl_ref, cb_ref, xl_out, ids_out, loss_ref) = refs

    hnew = h_ref[...]
    if first:
        xl_out[...] = hnew
    else:
        xl_out[...] = xl_ref[...] + hnew

    resid = hnew
    lsum = jnp.float32(0.0)
    idcols = []
    for r in range(_RES):
        cbn = _rownorm(cb_ref[0, r])
        rn = _rownorm(resid)
        sim = _mmT(rn, cbn)
        m = jnp.max(sim, axis=1, keepdims=True)
        io = lax.broadcasted_iota(_I32, sim.shape, 1)
        idx = jnp.min(jnp.where(sim >= m, io, _CODES), axis=1, keepdims=True)
        oh = (io == idx).astype(_F32)
        q = lax.dot_general(
            oh, cbn, (((1,), (0,)), ((), ())),
            preferred_element_type=_F32, precision=lax.Precision.HIGHEST)
        d = q - resid
        lsum = lsum + jnp.sum(d * d)
        idcols.append(idx)
        resid = resid - q
    ids_out[...] = jnp.concatenate(idcols, axis=1)

    @pl.when(pl.program_id(0) == 0)
    def _():
        loss_ref[...] = jnp.zeros((1, 1), _F32)
    loss_ref[...] += jnp.full((1, 1), lsum * jnp.float32(0.25 / (_N * _H)))


def _tc_head_body(xl_ref, wp_ref, bp_ref, wg_ref, bg_ref, pred_out, gnn_out):
    xl = xl_ref[...]
    pred_out[...] = _mmT(xl, wp_ref[...]) + bp_ref[...]
    gnn_out[...] = _mmT(xl, wg_ref[...]) + bg_ref[...]


def _row_spec(cols):
    return pl.BlockSpec((_BN, cols), lambda i: (i, 0))


def _const_spec(shape):
    nd = len(shape)
    return pl.BlockSpec(shape, lambda i, _n=nd: (0,) * _n)


def _layer_spec(shape, li):
    nd = len(shape)
    return pl.BlockSpec((1,) + shape, lambda i, _n=nd, _li=li: (_li,) + (0,) * _n)


def _lrow_spec(li):
    return pl.BlockSpec((1, 1, _H), lambda i, _li=li: (_li, 0, 0))


def _make_tc_update(li):
    return pl.pallas_call(
        _tc_update_body,
        grid=(_GRID,),
        in_specs=[
            _row_spec(_H),
            pl.BlockSpec((_NC, _BN, _H), lambda i: (0, i, 0)),
            pl.BlockSpec((_NC, _BN, 16), lambda i: (0, i, 0)),
            _layer_spec((_H, _H), li),
            _lrow_spec(li),
            _layer_spec((_H, _H), li),
            _layer_spec((_H, _H), li),
            _lrow_spec(li),
            _lrow_spec(li),
            _lrow_spec(li),
        ],
        out_specs=_row_spec(_H),
        out_shape=jax.ShapeDtypeStruct((_N, _H), _F32),
    )


def _make_tc_rvq(first, li):
    in_specs = [_row_spec(_H)]
    if not first:
        in_specs.append(_row_spec(_H))
    in_specs.append(_layer_spec((_RES, _CODES, _H), li))
    return pl.pallas_call(
        functools.partial(_tc_rvq_body, first),
        grid=(_GRID,),
        in_specs=in_specs,
        out_specs=[
            _row_spec(_H),
            _row_spec(_RES),
            pl.BlockSpec((1, 1), lambda i: (0, 0)),
        ],
        out_shape=[
            jax.ShapeDtypeStruct((_N, _H), _F32),
            jax.ShapeDtypeStruct((_N, _RES), _I32),
            jax.ShapeDtypeStruct((1, 1), _F32),
        ],
    )


def _make_tc_head():
    return pl.pallas_call(
        _tc_head_body,
        grid=(_GRID,),
        in_specs=[
            _row_spec(_H),
            _const_spec((_OUT, _H)),
            _const_spec((1, _OUT)),
            _const_spec((16, _H)),
            _const_spec((1, 16)),
        ],
        out_specs=[_row_spec(_OUT), _row_spec(16)],
        out_shape=[
            jax.ShapeDtypeStruct((_N, _OUT), _F32),
            jax.ShapeDtypeStruct((_N, 16), _F32),
        ],
    )


def kernel(x, edge_index, Wl, bl, Wr, Wlin, blin, bn_g, bn_b, codebooks,
           W_gnn, b_gnn, W_pred, b_pred):
    src = edge_index[0].reshape(_NW, _NCH, _CHUNK)
    dst = edge_index[1].reshape(_NW, _NCH, _CHUNK)
    z128 = jnp.zeros((_CHUNK, _H), _F32)
    one16 = jnp.ones((_CHUNK, 16), _F32)
    z16 = jnp.zeros((_CHUNK, 16), _F32)

    sc_agg = _make_sc_call()
    sc_deg = _make_deg_call()
    tc_head = _make_tc_head()

    wg_pad = jnp.zeros((16, _H), _F32).at[: _L * _RES].set(W_gnn)
    bg_pad = jnp.zeros((1, 16), _F32).at[0, : _L * _RES].set(b_gnn)

    bl3 = bl.reshape(_L, 1, _H)
    blin3 = blin.reshape(_L, 1, _H)
    bng3 = bn_g.reshape(_L, 1, _H)
    bnb3 = bn_b.reshape(_L, 1, _H)

    h = x
    xl = None
    losses = []
    ids = []
    deg2 = sc_deg(dst, one16, z16)
    for i in range(_L):
        agg2 = sc_agg(h, src, dst, z128)
        h = _make_tc_update(i)(h, agg2, deg2, Wl, bl3, Wr, Wlin, blin3,
                               bng3, bnb3)
        if i == 0:
            xl, ids_i, loss_i = _make_tc_rvq(True, i)(h, codebooks)
        else:
            xl, ids_i, loss_i = _make_tc_rvq(False, i)(h, xl, codebooks)
        losses.append(loss_i)
        ids.append(ids_i)
    pred, gnn_pad = tc_head(xl, W_pred, b_pred.reshape(1, _OUT), wg_pad, bg_pad)

    total_commit = (losses[0] + losses[1] + losses[2])[0, 0]
    id_cat = jnp.concatenate(ids, axis=1)
    gnn_id = gnn_pad[:, : _L * _RES]
    return (pred, total_commit, id_cat, gnn_id)


# speedup vs baseline: 1.2752x; 1.0399x over previous
import functools

import jax
import jax.numpy as jnp
from jax import lax
from jax.experimental import pallas as pl
from jax.experimental.pallas import tpu as pltpu
from jax.experimental.pallas import tpu_sc as plsc

_N = 10000
_E = 320000
_H = 128
_OUT = 40
_L = 3
_RES = 3
_CODES = 16

_NC = 2
_NS = 16
_NW = _NC * _NS
_EPW = _E // _NW
_CHUNK = 100
_NCH = _EPW // _CHUNK
_STRIPE = _N // _NS


def _stripe_pieces():
    out, off = [], 0
    while off < _STRIPE:
        size = min(_CHUNK, _STRIPE - off)
        out.append((off, size))
        off += size
    return out

_BN = 1000
_GRID = _N // _BN

_F32 = jnp.float32
_I32 = jnp.int32


def _sc_body(h_hbm, src_hbm, dst_hbm, z128_hbm, agg_hbm,
             src_v, dst_v, b0, b1, agg_sh, g0, g1, aux):
    cid = lax.axis_index("c")
    sid = lax.axis_index("s")
    wid = sid * _NC + cid
    base = sid * _STRIPE
    pieces = _stripe_pieces()

    pltpu.async_copy(src_hbm.at[wid], src_v, aux)
    pltpu.async_copy(dst_hbm.at[wid], dst_v, aux)
    pltpu.sync_copy(z128_hbm, b0)
    for off, size in pieces:
        pltpu.async_copy(b0.at[pl.ds(0, size)],
                         agg_sh.at[pl.ds(base + off, size)], aux)
    pltpu.make_async_copy(src_hbm.at[wid], src_v, aux).wait()
    pltpu.make_async_copy(dst_hbm.at[wid], dst_v, aux).wait()
    for off, size in pieces:
        pltpu.make_async_copy(b0.at[pl.ds(0, size)],
                              agg_sh.at[pl.ds(base + off, size)], aux).wait()
    plsc.subcore_barrier()

    pltpu.async_copy(h_hbm.at[src_v.at[0]], b0, g0)

    def pair(p, carry):
        a = 2 * p
        pltpu.async_copy(h_hbm.at[src_v.at[a + 1]], b1, g1)
        pltpu.make_async_copy(h_hbm.at[src_v.at[a]], b0, g0).wait()
        pltpu.sync_copy(b0, agg_sh.at[dst_v.at[a]], add=True)
        nxt = jnp.minimum(a + 2, _NCH - 1)
        pltpu.async_copy(h_hbm.at[src_v.at[nxt]], b0, g0)
        pltpu.make_async_copy(h_hbm.at[src_v.at[a + 1]], b1, g1).wait()
        pltpu.sync_copy(b1, agg_sh.at[dst_v.at[a + 1]], add=True)
        return carry

    lax.fori_loop(0, _NCH // 2, pair, 0)
    pltpu.make_async_copy(h_hbm.at[src_v.at[_NCH - 1]], b0, g0).wait()
    if _NCH % 2 == 1:
        pltpu.sync_copy(b0, agg_sh.at[dst_v.at[_NCH - 1]], add=True)
    plsc.subcore_barrier()

    pend = [None, None]
    for i, (off, size) in enumerate(pieces):
        par = i % 2
        buf, s = (b0, g0) if par == 0 else (b1, g1)
        if pend[par] is not None:
            poff, psize = pend[par]
            pltpu.make_async_copy(
                buf.at[pl.ds(0, psize)],
                agg_hbm.at[cid].at[pl.ds(base + poff, psize)], s).wait()
        pltpu.sync_copy(agg_sh.at[pl.ds(base + off, size)],
                        buf.at[pl.ds(0, size)])
        pltpu.async_copy(buf.at[pl.ds(0, size)],
                         agg_hbm.at[cid].at[pl.ds(base + off, size)], s)
        pend[par] = (off, size)
    for par in (0, 1):
        if pend[par] is not None:
            poff, psize = pend[par]
            buf, s = (b0, g0) if par == 0 else (b1, g1)
            pltpu.make_async_copy(
                buf.at[pl.ds(0, psize)],
                agg_hbm.at[cid].at[pl.ds(base + poff, psize)], s).wait()


def _make_sc_call():
    mesh = plsc.VectorSubcoreMesh(
        core_axis_name="c", subcore_axis_name="s",
        num_cores=_NC, num_subcores=_NS)
    scratch = [
        pltpu.VMEM((_NCH, _CHUNK), _I32),
        pltpu.VMEM((_NCH, _CHUNK), _I32),
        pltpu.VMEM((_CHUNK, _H), _F32),
        pltpu.VMEM((_CHUNK, _H), _F32),
        pltpu.VMEM_SHARED((_N, _H), _F32),
        pltpu.SemaphoreType.DMA,
        pltpu.SemaphoreType.DMA,
        pltpu.SemaphoreType.DMA,
    ]
    return pl.kernel(
        _sc_body,
        out_type=jax.ShapeDtypeStruct((_NC, _N, _H), _F32),
        mesh=mesh,
        scratch_types=scratch,
        compiler_params=pltpu.CompilerParams(use_tc_tiling_on_sc=False),
    )


def _deg_body(dst_hbm, one16_hbm, z16_hbm, deg_hbm,
              dst_v, ones_v, degrows_v, deg_sh, sem):
    cid = lax.axis_index("c")
    sid = lax.axis_index("s")
    wid = sid * _NC + cid
    base = sid * _STRIPE

    pltpu.sync_copy(dst_hbm.at[wid], dst_v)
    pltpu.sync_copy(one16_hbm, ones_v)
    pltpu.sync_copy(z16_hbm, degrows_v)
    for off, size in _stripe_pieces():
        pltpu.sync_copy(degrows_v.at[pl.ds(0, size)],
                        deg_sh.at[pl.ds(base + off, size)])
    plsc.subcore_barrier()

    def fire(j, carry):
        pltpu.async_copy(ones_v, deg_sh.at[dst_v.at[j]], sem, add=True)
        return carry

    lax.fori_loop(0, _NCH, fire, 0)

    def drain(j, carry):
        pltpu.make_async_copy(ones_v, deg_sh.at[dst_v.at[0]], sem).wait()
        return carry

    lax.fori_loop(0, _NCH, drain, 0)
    plsc.subcore_barrier()

    for off, size in _stripe_pieces():
        sl = pl.ds(base + off, size)
        pltpu.sync_copy(deg_sh.at[sl], degrows_v.at[pl.ds(0, size)])
        pltpu.sync_copy(degrows_v.at[pl.ds(0, size)], deg_hbm.at[cid].at[sl])


def _make_deg_call():
    mesh = plsc.VectorSubcoreMesh(
        core_axis_name="c", subcore_axis_name="s",
        num_cores=_NC, num_subcores=_NS)
    scratch = [
        pltpu.VMEM((_NCH, _CHUNK), _I32),
        pltpu.VMEM((_CHUNK, 16), _F32),
        pltpu.VMEM((_CHUNK, 16), _F32),
        pltpu.VMEM_SHARED((_N, 16), _F32),
        pltpu.SemaphoreType.DMA,
    ]
    return pl.kernel(
        _deg_body,
        out_type=jax.ShapeDtypeStruct((_NC, _N, 16), _F32),
        mesh=mesh,
        scratch_types=scratch,
        compiler_params=pltpu.CompilerParams(use_tc_tiling_on_sc=False),
    )


def _mmT(a, w):
    return lax.dot_general(
        a, w, (((1,), (1,)), ((), ())),
        preferred_element_type=_F32, precision=lax.Precision.DEFAULT)


def _rownorm(v):
    ss = jnp.sum(v * v, axis=1, keepdims=True)
    return v / jnp.maximum(jnp.sqrt(ss), 1e-12)


def _tc_update_body(h_ref, agg_ref, deg_ref, wl_ref, bl_ref, wr_ref,
                    wlin_ref, blin_ref, g_ref, b_ref, h_out):
    h = h_ref[...]
    agg = agg_ref[0] + agg_ref[1]
    deg = deg_ref[0, :, 0:1] + deg_ref[1, :, 0:1]
    aggn = agg * (1.0 / jnp.maximum(deg, 1.0))

    out = _mmT(aggn, wl_ref[0]) + bl_ref[0] + _mmT(h, wr_ref[0])
    z = _rownorm(out) + _mmT(h, wlin_ref[0]) + blin_ref[0]
    scale = g_ref[0] * (1.0 / jnp.sqrt(jnp.float32(1.0 + 1e-5)))
    h_out[...] = jnp.maximum(z * scale + b_ref[0], 0.0)


def _tc_rvq_body(first, last, *refs):
    if last:
        (h_ref, cb_ref, ids_out, loss_ref) = refs
    elif first:
        (h_ref, cb_ref, xl_out, ids_out, loss_ref) = refs
    else:
        (h_ref, xl_ref, cb_ref, xl_out, ids_out, loss_ref) = refs

    hnew = h_ref[...]
    if not last:
        if first:
            xl_out[...] = hnew
        else:
            xl_out[...] = xl_ref[...] + hnew

    resid = hnew
    lsum = jnp.float32(0.0)
    idcols = []
    for r in range(_RES):
        cbn = _rownorm(cb_ref[0, r])
        nq = jnp.sum(cbn * cbn, axis=1)[None, :]
        ssr = jnp.sum(resid * resid, axis=1, keepdims=True)
        nr = jnp.maximum(jnp.sqrt(ssr), 1e-12)
        rn = resid / nr
        sim = _mmT(rn, cbn)
        m = jnp.max(sim, axis=1, keepdims=True)
        io = lax.broadcasted_iota(_I32, sim.shape, 1)
        idx = jnp.min(jnp.where(sim >= m, io, _CODES), axis=1, keepdims=True)
        oh = (io == idx).astype(_F32)
        nq_row = jnp.sum(oh * nq, axis=1, keepdims=True)
        lsum = lsum + jnp.sum(ssr - 2.0 * m * nr + nq_row)
        idcols.append(idx)
        if r < _RES - 1:
            q = lax.dot_general(
                oh, cbn, (((1,), (0,)), ((), ())),
                preferred_element_type=_F32, precision=lax.Precision.HIGHEST)
            resid = resid - q
    ids_out[...] = jnp.concatenate(idcols, axis=1)

    @pl.when(pl.program_id(0) == 0)
    def _():
        loss_ref[...] = jnp.zeros((1, 1), _F32)
    loss_ref[...] += jnp.full((1, 1), lsum * jnp.float32(0.25 / (_N * _H)))


def _tc_head_body(xl_ref, h_ref, wp_ref, bp_ref, wg_ref, bg_ref,
                  pred_out, gnn_out):
    xl = xl_ref[...] + h_ref[...]
    pred_out[...] = _mmT(xl, wp_ref[...]) + bp_ref[...]
    gnn_out[...] = _mmT(xl, wg_ref[...]) + bg_ref[...]


def _row_spec(cols):
    return pl.BlockSpec((_BN, cols), lambda i: (i, 0))


def _const_spec(shape):
    nd = len(shape)
    return pl.BlockSpec(shape, lambda i, _n=nd: (0,) * _n)


def _layer_spec(shape, li):
    nd = len(shape)
    return pl.BlockSpec((1,) + shape, lambda i, _n=nd, _li=li: (_li,) + (0,) * _n)


def _lrow_spec(li):
    return pl.BlockSpec((1, 1, _H), lambda i, _li=li: (_li, 0, 0))


def _make_tc_update(li):
    return pl.pallas_call(
        _tc_update_body,
        grid=(_GRID,),
        in_specs=[
            _row_spec(_H),
            pl.BlockSpec((_NC, _BN, _H), lambda i: (0, i, 0)),
            pl.BlockSpec((_NC, _BN, 16), lambda i: (0, i, 0)),
            _layer_spec((_H, _H), li),
            _lrow_spec(li),
            _layer_spec((_H, _H), li),
            _layer_spec((_H, _H), li),
            _lrow_spec(li),
            _lrow_spec(li),
            _lrow_spec(li),
        ],
        out_specs=_row_spec(_H),
        out_shape=jax.ShapeDtypeStruct((_N, _H), _F32),
    )


def _make_tc_rvq(first, last, li):
    in_specs = [_row_spec(_H)]
    if not first and not last:
        in_specs.append(_row_spec(_H))
    in_specs.append(_layer_spec((_RES, _CODES, _H), li))
    out_specs = []
    out_shape = []
    if not last:
        out_specs.append(_row_spec(_H))
        out_shape.append(jax.ShapeDtypeStruct((_N, _H), _F32))
    out_specs += [
        _row_spec(_RES),
        pl.BlockSpec((1, 1), lambda i: (0, 0)),
    ]
    out_shape += [
        jax.ShapeDtypeStruct((_N, _RES), _I32),
        jax.ShapeDtypeStruct((1, 1), _F32),
    ]
    return pl.pallas_call(
        functools.partial(_tc_rvq_body, first, last),
        grid=(_GRID,),
        in_specs=in_specs,
        out_specs=out_specs,
        out_shape=out_shape,
    )


def _make_tc_head():
    return pl.pallas_call(
        _tc_head_body,
        grid=(_GRID,),
        in_specs=[
            _row_spec(_H),
            _row_spec(_H),
            _const_spec((_OUT, _H)),
            _const_spec((1, _OUT)),
            _const_spec((16, _H)),
            _const_spec((1, 16)),
        ],
        out_specs=[_row_spec(_OUT), _row_spec(16)],
        out_shape=[
            jax.ShapeDtypeStruct((_N, _OUT), _F32),
            jax.ShapeDtypeStruct((_N, 16), _F32),
        ],
    )


def kernel(x, edge_index, Wl, bl, Wr, Wlin, blin, bn_g, bn_b, codebooks,
           W_gnn, b_gnn, W_pred, b_pred):
    src = edge_index[0].reshape(_NW, _NCH, _CHUNK)
    dst = edge_index[1].reshape(_NW, _NCH, _CHUNK)
    z128 = jnp.zeros((_CHUNK, _H), _F32)
    one16 = jnp.ones((_CHUNK, 16), _F32)
    z16 = jnp.zeros((_CHUNK, 16), _F32)

    sc_agg = _make_sc_call()
    sc_deg = _make_deg_call()
    tc_head = _make_tc_head()

    wg_pad = jnp.zeros((16, _H), _F32).at[: _L * _RES].set(W_gnn)
    bg_pad = jnp.zeros((1, 16), _F32).at[0, : _L * _RES].set(b_gnn)

    bl3 = bl.reshape(_L, 1, _H)
    blin3 = blin.reshape(_L, 1, _H)
    bng3 = bn_g.reshape(_L, 1, _H)
    bnb3 = bn_b.reshape(_L, 1, _H)

    h = x
    xl = None
    losses = []
    ids = []
    deg2 = sc_deg(dst, one16, z16)
    for i in range(_L):
        agg2 = sc_agg(h, src, dst, z128)
        h = _make_tc_update(i)(h, agg2, deg2, Wl, bl3, Wr, Wlin, blin3,
                               bng3, bnb3)
        if i == 0:
            xl, ids_i, loss_i = _make_tc_rvq(True, False, i)(h, codebooks)
        elif i < _L - 1:
            xl, ids_i, loss_i = _make_tc_rvq(False, False, i)(h, xl, codebooks)
        else:
            ids_i, loss_i = _make_tc_rvq(False, True, i)(h, codebooks)
        losses.append(loss_i)
        ids.append(ids_i)
    pred, gnn_pad = tc_head(xl, h, W_pred, b_pred.reshape(1, _OUT), wg_pad,
                            bg_pad)

    total_commit = (losses[0] + losses[1] + losses[2])[0, 0]
    id_cat = jnp.concatenate(ids, axis=1)
    gnn_id = gnn_pad[:, : _L * _RES]
    return (pred, total_commit, id_cat, gnn_id)
